# baseline (speedup 1.0000x reference)
import functools

import jax
import jax.numpy as jnp
from jax import lax
from jax.experimental import pallas as pl
from jax.experimental.pallas import tpu as pltpu
from jax.experimental.pallas import tpu_sc as plsc

N = 10000
E = 320000
D = 128
T = 3
G = 256
EMB = 512
L = 4

NP_ = 10240
BLK = 512
NB = NP_ // BLK
NC = 2
NS = 16
NW = NC * NS
K = 64
CHC = 160
NCHT = NW * CHC
EP = NCHT * K
RPT = NP_ // NS


def _prep_body(src_ref, et_ref, out_ref):
    out_ref[...] = et_ref[...] * NP_ + src_ref[...]


def _edge_prep(src2d, et2d):
    return pl.pallas_call(
        _prep_body,
        out_shape=jax.ShapeDtypeStruct(src2d.shape, jnp.int32),
    )(src2d, et2d)


def _msg_body(h_ref, w_ref, b_ref, out_ref):
    out_ref[0] = (
        jnp.dot(h_ref[...], w_ref[0], preferred_element_type=jnp.float32)
        + b_ref[0, 0]
    )


def _msg(h, W_msg, bm):
    return pl.pallas_call(
        _msg_body,
        grid=(T, NB),
        in_specs=[
            pl.BlockSpec((BLK, D), lambda t, i: (i, 0)),
            pl.BlockSpec((1, D, D), lambda t, i: (t, 0, 0)),
            pl.BlockSpec((1, 1, D), lambda t, i: (t, 0, 0)),
        ],
        out_specs=pl.BlockSpec((1, BLK, D), lambda t, i: (t, i, 0)),
        out_shape=jax.ShapeDtypeStruct((T, NP_, D), jnp.float32),
    )(h, W_msg, bm)


def _sc_body(table, idx_hbm, zeros_hbm, out_hbm, idx_v, rows_v, agg_sh,
             si0, si1, si2, si3, si4, si5, si6, si7,
             sg0, sg1, sg2, sg3, ss0, ss1, ss2, ss3):
    c = lax.axis_index("c")
    s = lax.axis_index("s")
    base_chunk = c * (NS * CHC) + s
    sem_i = (si0, si1, si2, si3, si4, si5, si6, si7)
    sem_g = (sg0, sg1, sg2, sg3)
    sem_s = (ss0, ss1, ss2, ss3)

    with jax.named_scope("agg_zero"):
        pltpu.sync_copy(zeros_hbm, rows_v.at[0])
        base = s * RPT
        for k in range(RPT // K):
            pltpu.sync_copy(rows_v.at[0], agg_sh.at[pl.ds(base + k * K, K)])
        plsc.subcore_barrier()

    def start_idx(j, sl):
        pltpu.async_copy(idx_hbm.at[base_chunk + NS * j], idx_v.at[sl],
                         sem_i[sl])

    def wait_idx(sl):
        pltpu.make_async_copy(idx_hbm.at[0], idx_v.at[sl], sem_i[sl]).wait()

    def start_gather(b, sl):
        pltpu.async_copy(table.at[idx_v.at[sl, 0]], rows_v.at[b], sem_g[b])

    def wait_gather(b):
        pltpu.make_async_copy(table.at[pl.ds(0, K)], rows_v.at[b],
                              sem_g[b]).wait()

    def start_scatter(b, sl):
        pltpu.async_copy(rows_v.at[b], agg_sh.at[idx_v.at[sl, 1]], sem_s[b],
                         add=True)

    def wait_scatter(b):
        pltpu.make_async_copy(rows_v.at[b], agg_sh.at[pl.ds(0, K)],
                              sem_s[b]).wait()

    with jax.named_scope("edge_loop_prime"):
        for j in range(6):
            start_idx(j, j)
        wait_idx(0)
        start_gather(0, 0)
        wait_idx(1)
        start_gather(1, 1)

    @pl.loop(0, CHC, step=8)
    def _(g):
        for x in range(8):
            o, q = x % 4, x
            o2, q2 = (x + 2) % 4, (x + 2) % 8
            wait_gather(o)
            start_scatter(o, q)

            if x < 2:
                pl.when(g > 0)(lambda: wait_scatter(o2))
            else:
                wait_scatter(o2)

            @pl.when(g + x + 2 < CHC)
            def _():
                wait_idx(q2)
                start_gather(o2, q2)

            @pl.when(g + x + 6 < CHC)
            def _():
                start_idx(g + x + 6, (x + 6) % 8)

    wait_scatter(2)
    wait_scatter(3)

    with jax.named_scope("post_barrier"):
        plsc.subcore_barrier()

    with jax.named_scope("agg_writeout"):
        for k in range(RPT // K):
            pltpu.sync_copy(agg_sh.at[pl.ds(base + k * K, K)], rows_v.at[0])
            pltpu.sync_copy(rows_v.at[0], out_hbm.at[c, pl.ds(base + k * K, K)])


@functools.cache
def _make_sc_scatter():
    return pl.kernel(
        _sc_body,
        out_type=jax.ShapeDtypeStruct((NC, NP_, D), jnp.float32),
        mesh=plsc.VectorSubcoreMesh(core_axis_name="c", subcore_axis_name="s",
                                    num_cores=NC, num_subcores=NS),
        scratch_types=[
            pltpu.VMEM((8, 2, K), jnp.int32),
            pltpu.VMEM((4, K, D), jnp.float32),
            pltpu.VMEM_SHARED((NP_, D), jnp.float32),
        ] + [pltpu.SemaphoreType.DMA] * 16,
    )


def _sc_scatter(table, idx_pack, zeros128):
    return _make_sc_scatter()(table, idx_pack, zeros128)


def _gru(h_ref, agg_ref, wz, uz, bz, wr, ur, br, wh, uh, bh):
    h = h_ref[...]
    a = agg_ref[...].sum(axis=0)
    dot = lambda x, m: jnp.dot(x, m[...], preferred_element_type=jnp.float32)
    z = jax.nn.sigmoid(dot(a, wz) + dot(h, uz) + bz[...])
    r = jax.nn.sigmoid(dot(a, wr) + dot(h, ur) + br[...])
    ht = jnp.tanh(dot(a, wh) + dot(r * h, uh) + bh[...])
    return (1.0 - z) * h + z * ht


def _gru_msg_body(h_ref, agg_ref, wz, uz, bz, wr, ur, br, wh, uh, bh,
                  wm, bm, hn_ref, tr_ref):
    hn = _gru(h_ref, agg_ref, wz, uz, bz, wr, ur, br, wh, uh, bh)
    hn_ref[...] = hn
    for t in range(T):
        tr_ref[t] = (
            jnp.dot(hn, wm[t], preferred_element_type=jnp.float32) + bm[t, 0]
        )


def _gru_msg(h, aggs, Wz, Uz, bz, Wr, Ur, br, Wh, Uh, bh, W_msg, bm):
    full = lambda *blk: pl.BlockSpec(blk, lambda i: (0,) * len(blk))
    return pl.pallas_call(
        _gru_msg_body,
        grid=(NB,),
        in_specs=[
            pl.BlockSpec((BLK, D), lambda i: (i, 0)),
            pl.BlockSpec((NC, BLK, D), lambda i: (0, i, 0)),
            full(D, D), full(D, D), full(1, D),
            full(D, D), full(D, D), full(1, D),
            full(D, D), full(D, D), full(1, D),
            full(T, D, D), full(T, 1, D),
        ],
        out_specs=[
            pl.BlockSpec((BLK, D), lambda i: (i, 0)),
            pl.BlockSpec((T, BLK, D), lambda i: (0, i, 0)),
        ],
        out_shape=[
            jax.ShapeDtypeStruct((NP_, D), jnp.float32),
            jax.ShapeDtypeStruct((T, NP_, D), jnp.float32),
        ],
    )(h, aggs, Wz, Uz, bz, Wr, Ur, br, Wh, Uh, bh, W_msg, bm)


def _gru_readout_body(h_ref, agg_ref, wz, uz, bz, wr, ur, br, wh, uh, bh,
                      wu, bu, wg, bg, ids_ref, out_ref):
    hn = _gru(h_ref, agg_ref, wz, uz, bz, wr, ur, br, wh, uh, bh)
    up = jnp.dot(hn, wu[...], preferred_element_type=jnp.float32) + bu[...]
    gt = jax.nn.sigmoid(
        jnp.dot(hn, wg[...], preferred_element_type=jnp.float32) + bg[...]
    )
    y = gt * up
    ids = ids_ref[0, 0, :]
    onehot = (
        lax.broadcasted_iota(jnp.int32, (G, BLK), 0) == ids[None, :]
    ).astype(jnp.float32)
    contrib = jnp.dot(onehot, y, preferred_element_type=jnp.float32)

    @pl.when(pl.program_id(0) == 0)
    def _():
        out_ref[...] = jnp.zeros_like(out_ref)

    out_ref[...] += contrib


def _gru_readout(h, aggs, Wz, Uz, bz, Wr, Ur, br, Wh, Uh, bh,
                 Wu, bu, Wg, bg, ids):
    full = lambda *blk: pl.BlockSpec(blk, lambda i: (0,) * len(blk))
    return pl.pallas_call(
        _gru_readout_body,
        grid=(NB,),
        in_specs=[
            pl.BlockSpec((BLK, D), lambda i: (i, 0)),
            pl.BlockSpec((NC, BLK, D), lambda i: (0, i, 0)),
            full(D, D), full(D, D), full(1, D),
            full(D, D), full(D, D), full(1, D),
            full(D, D), full(D, D), full(1, D),
            full(D, EMB), full(1, EMB), full(D, EMB), full(1, EMB),
            pl.BlockSpec((1, 1, BLK), lambda i: (i, 0, 0)),
        ],
        out_specs=pl.BlockSpec((G, EMB), lambda i: (0, 0)),
        out_shape=jax.ShapeDtypeStruct((G, EMB), jnp.float32),
    )(h, aggs, Wz, Uz, bz, Wr, Ur, br, Wh, Uh, bh, Wu, bu, Wg, bg, ids)


def kernel(node_features, edge_index, edge_type, node_to_graph_id,
           W_msg, b_msg, Wz, Uz, bz, Wr, Ur, br, Wh, Uh, bh,
           Wu, bu, Wg, bg):
    f32 = jnp.float32
    h0 = jnp.pad(node_features, ((0, NP_ - N), (0, 0)))
    pad_src = jnp.arange(EP - E, dtype=jnp.int32) % N
    pad_dst = N + jnp.arange(EP - E, dtype=jnp.int32) % (NP_ - N)
    srcp = jnp.concatenate([edge_index[0], pad_src])
    dstp = jnp.concatenate([edge_index[1], pad_dst])
    etp = jnp.pad(edge_type, (0, EP - E))

    fused = _edge_prep(
        srcp.reshape(EP // K, K), etp.reshape(EP // K, K)
    ).reshape(NCHT, K)
    dst2d = dstp.reshape(NCHT, K)
    idx_pack = jnp.stack([fused, dst2d], axis=1)
    zeros128 = jnp.zeros((K, D), f32)
    idsp = jnp.pad(
        node_to_graph_id, (0, NP_ - N), constant_values=G
    ).reshape(NB, 1, BLK)

    bz2 = bz.reshape(1, D)
    br2 = br.reshape(1, D)
    bh2 = bh.reshape(1, D)
    bm2 = b_msg.reshape(T, 1, D)
    bu2 = bu.reshape(1, EMB)
    bg2 = bg.reshape(1, EMB)

    tr = _msg(h0, W_msg, bm2)
    h = h0
    for layer in range(L):
        aggs = _sc_scatter(tr.reshape(T * NP_, D), idx_pack, zeros128)
        if layer < L - 1:
            h, tr = _gru_msg(h, aggs, Wz, Uz, bz2, Wr, Ur, br2,
                             Wh, Uh, bh2, W_msg, bm2)
        else:
            out = _gru_readout(h, aggs, Wz, Uz, bz2, Wr, Ur, br2,
                               Wh, Uh, bh2, Wu, bu2, Wg, bg2, idsp)
    return out


# speedup vs baseline: 1.0373x; 1.0373x over previous
import functools

import jax
import jax.numpy as jnp
from jax import lax
from jax.experimental import pallas as pl
from jax.experimental.pallas import tpu as pltpu
from jax.experimental.pallas import tpu_sc as plsc

N = 10000
E = 320000
D = 128
T = 3
G = 256
EMB = 512
L = 4

NP_ = 10240
BLK = 512
NB = NP_ // BLK
NC = 2
NS = 16
NW = NC * NS
K = 80
CHC = 128
NCHT = NW * CHC
EP = NCHT * K
RPT = NP_ // NS


def _prep_body(src_ref, et_ref, out_ref):
    out_ref[...] = et_ref[...] * NP_ + src_ref[...]


def _edge_prep(src2d, et2d):
    return pl.pallas_call(
        _prep_body,
        out_shape=jax.ShapeDtypeStruct(src2d.shape, jnp.int32),
    )(src2d, et2d)


def _msg_body(h_ref, w_ref, b_ref, out_ref):
    out_ref[0] = (
        jnp.dot(h_ref[...], w_ref[0], preferred_element_type=jnp.float32)
        + b_ref[0, 0]
    )


def _msg(h, W_msg, bm):
    return pl.pallas_call(
        _msg_body,
        grid=(T, NB),
        in_specs=[
            pl.BlockSpec((BLK, D), lambda t, i: (i, 0)),
            pl.BlockSpec((1, D, D), lambda t, i: (t, 0, 0)),
            pl.BlockSpec((1, 1, D), lambda t, i: (t, 0, 0)),
        ],
        out_specs=pl.BlockSpec((1, BLK, D), lambda t, i: (t, i, 0)),
        out_shape=jax.ShapeDtypeStruct((T, NP_, D), jnp.float32),
    )(h, W_msg, bm)


def _sc_body(table, idx_hbm, zeros_hbm, out_hbm, idx_v, rows_v, agg_sh,
             si0, si1, si2, si3, si4, si5, si6, si7,
             sg0, sg1, sg2, sg3, ss0, ss1, ss2, ss3):
    c = lax.axis_index("c")
    s = lax.axis_index("s")
    base_chunk = c * (NS * CHC) + s
    sem_i = (si0, si1, si2, si3, si4, si5, si6, si7)
    sem_g = (sg0, sg1, sg2, sg3)
    sem_s = (ss0, ss1, ss2, ss3)

    with jax.named_scope("agg_zero"):
        pltpu.sync_copy(zeros_hbm, rows_v.at[0])
        base = s * RPT
        for k in range(RPT // K):
            pltpu.sync_copy(rows_v.at[0], agg_sh.at[pl.ds(base + k * K, K)])
        plsc.subcore_barrier()

    def start_idx(j, sl):
        pltpu.async_copy(idx_hbm.at[base_chunk + NS * j], idx_v.at[sl],
                         sem_i[sl])

    def wait_idx(sl):
        pltpu.make_async_copy(idx_hbm.at[0], idx_v.at[sl], sem_i[sl]).wait()

    def start_gather(b, sl):
        pltpu.async_copy(table.at[idx_v.at[sl, 0]], rows_v.at[b], sem_g[b])

    def wait_gather(b):
        pltpu.make_async_copy(table.at[pl.ds(0, K)], rows_v.at[b],
                              sem_g[b]).wait()

    def start_scatter(b, sl):
        pltpu.async_copy(rows_v.at[b], agg_sh.at[idx_v.at[sl, 1]], sem_s[b],
                         add=True)

    def wait_scatter(b):
        pltpu.make_async_copy(rows_v.at[b], agg_sh.at[pl.ds(0, K)],
                              sem_s[b]).wait()

    with jax.named_scope("edge_loop_prime"):
        for j in range(6):
            start_idx(j, j)
        wait_idx(0)
        start_gather(0, 0)
        wait_idx(1)
        start_gather(1, 1)

    @pl.loop(0, CHC, step=8)
    def _(g):
        for x in range(8):
            o, q = x % 4, x
            o2, q2 = (x + 2) % 4, (x + 2) % 8
            wait_gather(o)
            start_scatter(o, q)

            if x < 2:
                pl.when(g > 0)(lambda: wait_scatter(o2))
            else:
                wait_scatter(o2)

            @pl.when(g + x + 2 < CHC)
            def _():
                wait_idx(q2)
                start_gather(o2, q2)

            @pl.when(g + x + 6 < CHC)
            def _():
                start_idx(g + x + 6, (x + 6) % 8)

    wait_scatter(2)
    wait_scatter(3)

    with jax.named_scope("post_barrier"):
        plsc.subcore_barrier()

    with jax.named_scope("agg_writeout"):
        for k in range(RPT // K):
            pltpu.sync_copy(agg_sh.at[pl.ds(base + k * K, K)], rows_v.at[0])
            pltpu.sync_copy(rows_v.at[0], out_hbm.at[c, pl.ds(base + k * K, K)])


@functools.cache
def _make_sc_scatter():
    return pl.kernel(
        _sc_body,
        out_type=jax.ShapeDtypeStruct((NC, NP_, D), jnp.float32),
        mesh=plsc.VectorSubcoreMesh(core_axis_name="c", subcore_axis_name="s",
                                    num_cores=NC, num_subcores=NS),
        scratch_types=[
            pltpu.VMEM((8, 2, K), jnp.int32),
            pltpu.VMEM((4, K, D), jnp.float32),
            pltpu.VMEM_SHARED((NP_, D), jnp.float32),
        ] + [pltpu.SemaphoreType.DMA] * 16,
    )


def _sc_scatter(table, idx_pack, zeros128):
    return _make_sc_scatter()(table, idx_pack, zeros128)


def _gru(h_ref, agg_ref, wz, uz, bz, wr, ur, br, wh, uh, bh):
    h = h_ref[...]
    a = agg_ref[...].sum(axis=0)
    dot = lambda x, m: jnp.dot(x, m[...], preferred_element_type=jnp.float32)
    z = jax.nn.sigmoid(dot(a, wz) + dot(h, uz) + bz[...])
    r = jax.nn.sigmoid(dot(a, wr) + dot(h, ur) + br[...])
    ht = jnp.tanh(dot(a, wh) + dot(r * h, uh) + bh[...])
    return (1.0 - z) * h + z * ht


def _gru_msg_body(h_ref, agg_ref, wz, uz, bz, wr, ur, br, wh, uh, bh,
                  wm, bm, hn_ref, tr_ref):
    hn = _gru(h_ref, agg_ref, wz, uz, bz, wr, ur, br, wh, uh, bh)
    hn_ref[...] = hn
    for t in range(T):
        tr_ref[t] = (
            jnp.dot(hn, wm[t], preferred_element_type=jnp.float32) + bm[t, 0]
        )


def _gru_msg(h, aggs, Wz, Uz, bz, Wr, Ur, br, Wh, Uh, bh, W_msg, bm):
    full = lambda *blk: pl.BlockSpec(blk, lambda i: (0,) * len(blk))
    return pl.pallas_call(
        _gru_msg_body,
        grid=(NB,),
        in_specs=[
            pl.BlockSpec((BLK, D), lambda i: (i, 0)),
            pl.BlockSpec((NC, BLK, D), lambda i: (0, i, 0)),
            full(D, D), full(D, D), full(1, D),
            full(D, D), full(D, D), full(1, D),
            full(D, D), full(D, D), full(1, D),
            full(T, D, D), full(T, 1, D),
        ],
        out_specs=[
            pl.BlockSpec((BLK, D), lambda i: (i, 0)),
            pl.BlockSpec((T, BLK, D), lambda i: (0, i, 0)),
        ],
        out_shape=[
            jax.ShapeDtypeStruct((NP_, D), jnp.float32),
            jax.ShapeDtypeStruct((T, NP_, D), jnp.float32),
        ],
    )(h, aggs, Wz, Uz, bz, Wr, Ur, br, Wh, Uh, bh, W_msg, bm)


def _gru_readout_body(h_ref, agg_ref, wz, uz, bz, wr, ur, br, wh, uh, bh,
                      wu, bu, wg, bg, ids_ref, out_ref):
    hn = _gru(h_ref, agg_ref, wz, uz, bz, wr, ur, br, wh, uh, bh)
    up = jnp.dot(hn, wu[...], preferred_element_type=jnp.float32) + bu[...]
    gt = jax.nn.sigmoid(
        jnp.dot(hn, wg[...], preferred_element_type=jnp.float32) + bg[...]
    )
    y = gt * up
    ids = ids_ref[0, 0, :]
    onehot = (
        lax.broadcasted_iota(jnp.int32, (G, BLK), 0) == ids[None, :]
    ).astype(jnp.float32)
    contrib = jnp.dot(onehot, y, preferred_element_type=jnp.float32)

    @pl.when(pl.program_id(0) == 0)
    def _():
        out_ref[...] = jnp.zeros_like(out_ref)

    out_ref[...] += contrib


def _gru_readout(h, aggs, Wz, Uz, bz, Wr, Ur, br, Wh, Uh, bh,
                 Wu, bu, Wg, bg, ids):
    full = lambda *blk: pl.BlockSpec(blk, lambda i: (0,) * len(blk))
    return pl.pallas_call(
        _gru_readout_body,
        grid=(NB,),
        in_specs=[
            pl.BlockSpec((BLK, D), lambda i: (i, 0)),
            pl.BlockSpec((NC, BLK, D), lambda i: (0, i, 0)),
            full(D, D), full(D, D), full(1, D),
            full(D, D), full(D, D), full(1, D),
            full(D, D), full(D, D), full(1, D),
            full(D, EMB), full(1, EMB), full(D, EMB), full(1, EMB),
            pl.BlockSpec((1, 1, BLK), lambda i: (i, 0, 0)),
        ],
        out_specs=pl.BlockSpec((G, EMB), lambda i: (0, 0)),
        out_shape=jax.ShapeDtypeStruct((G, EMB), jnp.float32),
    )(h, aggs, Wz, Uz, bz, Wr, Ur, br, Wh, Uh, bh, Wu, bu, Wg, bg, ids)


def kernel(node_features, edge_index, edge_type, node_to_graph_id,
           W_msg, b_msg, Wz, Uz, bz, Wr, Ur, br, Wh, Uh, bh,
           Wu, bu, Wg, bg):
    f32 = jnp.float32
    h0 = jnp.pad(node_features, ((0, NP_ - N), (0, 0)))
    pad_src = jnp.arange(EP - E, dtype=jnp.int32) % N
    pad_dst = N + jnp.arange(EP - E, dtype=jnp.int32) % (NP_ - N)
    srcp = jnp.concatenate([edge_index[0], pad_src])
    dstp = jnp.concatenate([edge_index[1], pad_dst])
    etp = jnp.pad(edge_type, (0, EP - E))

    fused = _edge_prep(
        srcp.reshape(EP // K, K), etp.reshape(EP // K, K)
    ).reshape(NCHT, K)
    dst2d = dstp.reshape(NCHT, K)
    idx_pack = jnp.stack([fused, dst2d], axis=1)
    zeros128 = jnp.zeros((K, D), f32)
    idsp = jnp.pad(
        node_to_graph_id, (0, NP_ - N), constant_values=G
    ).reshape(NB, 1, BLK)

    bz2 = bz.reshape(1, D)
    br2 = br.reshape(1, D)
    bh2 = bh.reshape(1, D)
    bm2 = b_msg.reshape(T, 1, D)
    bu2 = bu.reshape(1, EMB)
    bg2 = bg.reshape(1, EMB)

    tr = _msg(h0, W_msg, bm2)
    h = h0
    for layer in range(L):
        aggs = _sc_scatter(tr.reshape(T * NP_, D), idx_pack, zeros128)
        if layer < L - 1:
            h, tr = _gru_msg(h, aggs, Wz, Uz, bz2, Wr, Ur, br2,
                             Wh, Uh, bh2, W_msg, bm2)
        else:
            out = _gru_readout(h, aggs, Wz, Uz, bz2, Wr, Ur, br2,
                               Wh, Uh, bh2, Wu, bu2, Wg, bg2, idsp)
    return out


# speedup vs baseline: 1.1063x; 1.0665x over previous
import functools

import jax
import jax.numpy as jnp
from jax import lax
from jax.experimental import pallas as pl
from jax.experimental.pallas import tpu as pltpu
from jax.experimental.pallas import tpu_sc as plsc

N = 10000
E = 320000
D = 128
T = 3
G = 256
EMB = 512
L = 4

NP_ = 10240
BLK = 512
NB = NP_ // BLK
NC = 2
NS = 16
NW = NC * NS
K = 128
CHC = 80
NCHT = NW * CHC
EP = NCHT * K
RPT = NP_ // NS


def _prep_body(src_ref, et_ref, out_ref):
    out_ref[...] = et_ref[...] * NP_ + src_ref[...]


def _edge_prep(src2d, et2d):
    return pl.pallas_call(
        _prep_body,
        out_shape=jax.ShapeDtypeStruct(src2d.shape, jnp.int32),
    )(src2d, et2d)


def _msg_body(h_ref, w_ref, b_ref, out_ref):
    out_ref[0] = (
        jnp.dot(h_ref[...], w_ref[0], preferred_element_type=jnp.float32)
        + b_ref[0, 0]
    )


def _msg(h, W_msg, bm):
    return pl.pallas_call(
        _msg_body,
        grid=(T, NB),
        in_specs=[
            pl.BlockSpec((BLK, D), lambda t, i: (i, 0)),
            pl.BlockSpec((1, D, D), lambda t, i: (t, 0, 0)),
            pl.BlockSpec((1, 1, D), lambda t, i: (t, 0, 0)),
        ],
        out_specs=pl.BlockSpec((1, BLK, D), lambda t, i: (t, i, 0)),
        out_shape=jax.ShapeDtypeStruct((T, NP_, D), jnp.float32),
    )(h, W_msg, bm)


def _sc_body(table, idx_hbm, zeros_hbm, out_hbm,
             idx_v, rows_v, agg_sh, sem_i0, sem_i1, sem_i2, sem_i3,
             sem_g0, sem_g1):
    c = lax.axis_index("c")
    s = lax.axis_index("s")
    base_chunk = c * (NS * CHC) + s
    sem_i = (sem_i0, sem_i1, sem_i2, sem_i3)
    sem_g = (sem_g0, sem_g1)

    with jax.named_scope("agg_zero"):
        pltpu.sync_copy(zeros_hbm, rows_v.at[0])
        base = s * RPT
        for k in range(RPT // K):
            pltpu.sync_copy(rows_v.at[0], agg_sh.at[pl.ds(base + k * K, K)])
        plsc.subcore_barrier()

    def start_idx(j, sl):
        pltpu.async_copy(idx_hbm.at[base_chunk + NS * j], idx_v.at[sl],
                         sem_i[sl])

    def wait_idx(sl):
        pltpu.make_async_copy(idx_hbm.at[0], idx_v.at[sl], sem_i[sl]).wait()

    def start_gather(b, sl):
        pltpu.async_copy(table.at[idx_v.at[sl, 0]], rows_v.at[b], sem_g[b])

    def wait_gather(b):
        pltpu.make_async_copy(table.at[pl.ds(0, K)], rows_v.at[b],
                              sem_g[b]).wait()

    def scatter(b, sl):
        pltpu.sync_copy(rows_v.at[b], agg_sh.at[idx_v.at[sl, 1]], add=True)

    with jax.named_scope("edge_loop_prime"):
        for j in range(4):
            start_idx(j, j)
        wait_idx(0)
        start_gather(0, 0)

    @pl.loop(0, CHC, step=4)
    def _(g):
        wait_idx(1)
        start_gather(1, 1)
        wait_gather(0)
        scatter(0, 0)

        @pl.when(g + 4 < CHC)
        def _():
            start_idx(g + 4, 0)

        wait_idx(2)
        start_gather(0, 2)
        wait_gather(1)
        scatter(1, 1)

        @pl.when(g + 5 < CHC)
        def _():
            start_idx(g + 5, 1)

        wait_idx(3)
        start_gather(1, 3)
        wait_gather(0)
        scatter(0, 2)

        @pl.when(g + 6 < CHC)
        def _():
            start_idx(g + 6, 2)

        @pl.when(g + 4 < CHC)
        def _():
            wait_idx(0)
            start_gather(0, 0)

        wait_gather(1)
        scatter(1, 3)

        @pl.when(g + 7 < CHC)
        def _():
            start_idx(g + 7, 3)

    with jax.named_scope("post_barrier"):
        plsc.subcore_barrier()

    with jax.named_scope("agg_writeout"):
        for k in range(RPT // K):
            pltpu.sync_copy(agg_sh.at[pl.ds(base + k * K, K)], rows_v.at[0])
            pltpu.sync_copy(rows_v.at[0], out_hbm.at[c, pl.ds(base + k * K, K)])


@functools.cache
def _make_sc_scatter():
    return pl.kernel(
        _sc_body,
        out_type=jax.ShapeDtypeStruct((NC, NP_, D), jnp.float32),
        mesh=plsc.VectorSubcoreMesh(core_axis_name="c", subcore_axis_name="s",
                                    num_cores=NC, num_subcores=NS),
        scratch_types=[
            pltpu.VMEM((4, 2, K), jnp.int32),
            pltpu.VMEM((2, K, D), jnp.float32),
            pltpu.VMEM_SHARED((NP_, D), jnp.float32),
        ] + [pltpu.SemaphoreType.DMA] * 6,
    )


def _sc_scatter(table, idx_pack, zeros128):
    return _make_sc_scatter()(table, idx_pack, zeros128)


def _gru(h_ref, agg_ref, wz, uz, bz, wr, ur, br, wh, uh, bh):
    h = h_ref[...]
    a = agg_ref[...].sum(axis=0)
    dot = lambda x, m: jnp.dot(x, m[...], preferred_element_type=jnp.float32)
    z = jax.nn.sigmoid(dot(a, wz) + dot(h, uz) + bz[...])
    r = jax.nn.sigmoid(dot(a, wr) + dot(h, ur) + br[...])
    ht = jnp.tanh(dot(a, wh) + dot(r * h, uh) + bh[...])
    return (1.0 - z) * h + z * ht


def _gru_msg_body(h_ref, agg_ref, wz, uz, bz, wr, ur, br, wh, uh, bh,
                  wm, bm, hn_ref, tr_ref):
    hn = _gru(h_ref, agg_ref, wz, uz, bz, wr, ur, br, wh, uh, bh)
    hn_ref[...] = hn
    for t in range(T):
        tr_ref[t] = (
            jnp.dot(hn, wm[t], preferred_element_type=jnp.float32) + bm[t, 0]
        )


def _gru_msg(h, aggs, Wz, Uz, bz, Wr, Ur, br, Wh, Uh, bh, W_msg, bm):
    full = lambda *blk: pl.BlockSpec(blk, lambda i: (0,) * len(blk))
    return pl.pallas_call(
        _gru_msg_body,
        grid=(NB,),
        in_specs=[
            pl.BlockSpec((BLK, D), lambda i: (i, 0)),
            pl.BlockSpec((NC, BLK, D), lambda i: (0, i, 0)),
            full(D, D), full(D, D), full(1, D),
            full(D, D), full(D, D), full(1, D),
            full(D, D), full(D, D), full(1, D),
            full(T, D, D), full(T, 1, D),
        ],
        out_specs=[
            pl.BlockSpec((BLK, D), lambda i: (i, 0)),
            pl.BlockSpec((T, BLK, D), lambda i: (0, i, 0)),
        ],
        out_shape=[
            jax.ShapeDtypeStruct((NP_, D), jnp.float32),
            jax.ShapeDtypeStruct((T, NP_, D), jnp.float32),
        ],
    )(h, aggs, Wz, Uz, bz, Wr, Ur, br, Wh, Uh, bh, W_msg, bm)


def _gru_readout_body(h_ref, agg_ref, wz, uz, bz, wr, ur, br, wh, uh, bh,
                      wu, bu, wg, bg, ids_ref, out_ref):
    hn = _gru(h_ref, agg_ref, wz, uz, bz, wr, ur, br, wh, uh, bh)
    up = jnp.dot(hn, wu[...], preferred_element_type=jnp.float32) + bu[...]
    gt = jax.nn.sigmoid(
        jnp.dot(hn, wg[...], preferred_element_type=jnp.float32) + bg[...]
    )
    y = gt * up
    ids = ids_ref[0, 0, :]
    onehot = (
        lax.broadcasted_iota(jnp.int32, (G, BLK), 0) == ids[None, :]
    ).astype(jnp.float32)
    contrib = jnp.dot(onehot, y, preferred_element_type=jnp.float32)

    @pl.when(pl.program_id(0) == 0)
    def _():
        out_ref[...] = jnp.zeros_like(out_ref)

    out_ref[...] += contrib


def _gru_readout(h, aggs, Wz, Uz, bz, Wr, Ur, br, Wh, Uh, bh,
                 Wu, bu, Wg, bg, ids):
    full = lambda *blk: pl.BlockSpec(blk, lambda i: (0,) * len(blk))
    return pl.pallas_call(
        _gru_readout_body,
        grid=(NB,),
        in_specs=[
            pl.BlockSpec((BLK, D), lambda i: (i, 0)),
            pl.BlockSpec((NC, BLK, D), lambda i: (0, i, 0)),
            full(D, D), full(D, D), full(1, D),
            full(D, D), full(D, D), full(1, D),
            full(D, D), full(D, D), full(1, D),
            full(D, EMB), full(1, EMB), full(D, EMB), full(1, EMB),
            pl.BlockSpec((1, 1, BLK), lambda i: (i, 0, 0)),
        ],
        out_specs=pl.BlockSpec((G, EMB), lambda i: (0, 0)),
        out_shape=jax.ShapeDtypeStruct((G, EMB), jnp.float32),
    )(h, aggs, Wz, Uz, bz, Wr, Ur, br, Wh, Uh, bh, Wu, bu, Wg, bg, ids)


def kernel(node_features, edge_index, edge_type, node_to_graph_id,
           W_msg, b_msg, Wz, Uz, bz, Wr, Ur, br, Wh, Uh, bh,
           Wu, bu, Wg, bg):
    f32 = jnp.float32
    h0 = jnp.pad(node_features, ((0, NP_ - N), (0, 0)))
    pad_src = jnp.arange(EP - E, dtype=jnp.int32) % N
    pad_dst = N + jnp.arange(EP - E, dtype=jnp.int32) % (NP_ - N)
    srcp = jnp.concatenate([edge_index[0], pad_src])
    dstp = jnp.concatenate([edge_index[1], pad_dst])
    etp = jnp.pad(edge_type, (0, EP - E))

    fused = _edge_prep(
        srcp.reshape(EP // K, K), etp.reshape(EP // K, K)
    ).reshape(NCHT, K)
    dst2d = dstp.reshape(NCHT, K)
    idx_pack = jnp.stack([fused, dst2d], axis=1)
    zeros128 = jnp.zeros((K, D), f32)
    idsp = jnp.pad(
        node_to_graph_id, (0, NP_ - N), constant_values=G
    ).reshape(NB, 1, BLK)

    bz2 = bz.reshape(1, D)
    br2 = br.reshape(1, D)
    bh2 = bh.reshape(1, D)
    bm2 = b_msg.reshape(T, 1, D)
    bu2 = bu.reshape(1, EMB)
    bg2 = bg.reshape(1, EMB)

    tr = _msg(h0, W_msg, bm2)
    h = h0
    for layer in range(L):
        aggs = _sc_scatter(tr.reshape(T * NP_, D), idx_pack, zeros128)
        if layer < L - 1:
            h, tr = _gru_msg(h, aggs, Wz, Uz, bz2, Wr, Ur, br2,
                             Wh, Uh, bh2, W_msg, bm2)
        else:
            out = _gru_readout(h, aggs, Wz, Uz, bz2, Wr, Ur, br2,
                               Wh, Uh, bh2, Wu, bu2, Wg, bg2, idsp)
    return out


# speedup vs baseline: 1.1639x; 1.0520x over previous
import functools

import jax
import jax.numpy as jnp
from jax import lax
from jax.experimental import pallas as pl
from jax.experimental.pallas import tpu as pltpu
from jax.experimental.pallas import tpu_sc as plsc

N = 10000
E = 320000
D = 128
T = 3
G = 256
EMB = 512
L = 4

NP_ = 10240
BLK = 512
NB = NP_ // BLK
NC = 2
NS = 16
NW = NC * NS
K = 128
CHC = 80
NCHT = NW * CHC
EP = NCHT * K
RPT = NP_ // NS


def _prep_body(src_ref, et_ref, out_ref):
    out_ref[...] = et_ref[...] * NP_ + src_ref[...]


def _edge_prep(src2d, et2d):
    return pl.pallas_call(
        _prep_body,
        out_shape=jax.ShapeDtypeStruct(src2d.shape, jnp.int32),
    )(src2d, et2d)


def _msg_body(h_ref, w_ref, b_ref, out_ref):
    prod = (
        jnp.dot(h_ref[...], w_ref[...], preferred_element_type=jnp.float32)
        + b_ref[...]
    )
    for t in range(T):
        out_ref[t] = prod[:, t * D:(t + 1) * D]


def _msg(h, Wmc, bmc):
    return pl.pallas_call(
        _msg_body,
        grid=(NB,),
        in_specs=[
            pl.BlockSpec((BLK, D), lambda i: (i, 0)),
            pl.BlockSpec((D, T * D), lambda i: (0, 0)),
            pl.BlockSpec((1, T * D), lambda i: (0, 0)),
        ],
        out_specs=pl.BlockSpec((T, BLK, D), lambda i: (0, i, 0)),
        out_shape=jax.ShapeDtypeStruct((T, NP_, D), jnp.float32),
    )(h, Wmc, bmc)


def _sc_body(table, idx_hbm, zeros_hbm, out_hbm,
             idx_v, rows_v, agg_sh, sem_i0, sem_i1, sem_i2, sem_i3,
             sem_g0, sem_g1):
    c = lax.axis_index("c")
    s = lax.axis_index("s")
    base_chunk = c * (NS * CHC) + s
    sem_i = (sem_i0, sem_i1, sem_i2, sem_i3)
    sem_g = (sem_g0, sem_g1)

    with jax.named_scope("agg_zero"):
        pltpu.sync_copy(zeros_hbm, rows_v.at[0])
        base = s * RPT
        for k in range(RPT // K):
            pltpu.sync_copy(rows_v.at[0], agg_sh.at[pl.ds(base + k * K, K)])
        plsc.subcore_barrier()

    def start_idx(j, sl):
        pltpu.async_copy(idx_hbm.at[base_chunk + NS * j], idx_v.at[sl],
                         sem_i[sl])

    def wait_idx(sl):
        pltpu.make_async_copy(idx_hbm.at[0], idx_v.at[sl], sem_i[sl]).wait()

    def start_gather(b, sl):
        pltpu.async_copy(table.at[idx_v.at[sl, 0]], rows_v.at[b], sem_g[b])

    def wait_gather(b):
        pltpu.make_async_copy(table.at[pl.ds(0, K)], rows_v.at[b],
                              sem_g[b]).wait()

    def scatter(b, sl):
        pltpu.sync_copy(rows_v.at[b], agg_sh.at[idx_v.at[sl, 1]], add=True)

    with jax.named_scope("edge_loop_prime"):
        for j in range(4):
            start_idx(j, j)
        wait_idx(0)
        start_gather(0, 0)

    @pl.loop(0, CHC, step=4)
    def _(g):
        wait_idx(1)
        start_gather(1, 1)
        wait_gather(0)
        scatter(0, 0)

        @pl.when(g + 4 < CHC)
        def _():
            start_idx(g + 4, 0)

        wait_idx(2)
        start_gather(0, 2)
        wait_gather(1)
        scatter(1, 1)

        @pl.when(g + 5 < CHC)
        def _():
            start_idx(g + 5, 1)

        wait_idx(3)
        start_gather(1, 3)
        wait_gather(0)
        scatter(0, 2)

        @pl.when(g + 6 < CHC)
        def _():
            start_idx(g + 6, 2)

        @pl.when(g + 4 < CHC)
        def _():
            wait_idx(0)
            start_gather(0, 0)

        wait_gather(1)
        scatter(1, 3)

        @pl.when(g + 7 < CHC)
        def _():
            start_idx(g + 7, 3)

    with jax.named_scope("post_barrier"):
        plsc.subcore_barrier()

    with jax.named_scope("agg_writeout"):
        for k in range(RPT // K):
            pltpu.sync_copy(agg_sh.at[pl.ds(base + k * K, K)], rows_v.at[0])
            pltpu.sync_copy(rows_v.at[0], out_hbm.at[c, pl.ds(base + k * K, K)])


@functools.cache
def _make_sc_scatter():
    return pl.kernel(
        _sc_body,
        out_type=jax.ShapeDtypeStruct((NC, NP_, D), jnp.float32),
        mesh=plsc.VectorSubcoreMesh(core_axis_name="c", subcore_axis_name="s",
                                    num_cores=NC, num_subcores=NS),
        scratch_types=[
            pltpu.VMEM((4, 2, K), jnp.int32),
            pltpu.VMEM((2, K, D), jnp.float32),
            pltpu.VMEM_SHARED((NP_, D), jnp.float32),
        ] + [pltpu.SemaphoreType.DMA] * 6,
    )


def _sc_scatter(table, idx_pack, zeros128):
    return _make_sc_scatter()(table, idx_pack, zeros128)


def _gru(h_ref, agg_ref, wzrh, uzr, uh, bzrh):
    h = h_ref[...]
    a = agg_ref[...].sum(axis=0)
    dot = lambda x, m: jnp.dot(x, m, preferred_element_type=jnp.float32)
    za = dot(a, wzrh[...]) + bzrh[...]
    zh = dot(h, uzr[...])
    z = jax.nn.sigmoid(za[:, :D] + zh[:, :D])
    r = jax.nn.sigmoid(za[:, D:2 * D] + zh[:, D:2 * D])
    ht = jnp.tanh(za[:, 2 * D:] + dot(r * h, uh[...]))
    return (1.0 - z) * h + z * ht


def _gru_msg_body(h_ref, agg_ref, wzrh, uzr, uh, bzrh, wmc, bmc,
                  hn_ref, tr_ref):
    hn = _gru(h_ref, agg_ref, wzrh, uzr, uh, bzrh)
    hn_ref[...] = hn
    prod = jnp.dot(hn, wmc[...], preferred_element_type=jnp.float32) + bmc[...]
    for t in range(T):
        tr_ref[t] = prod[:, t * D:(t + 1) * D]


def _gru_msg(h, aggs, Wzrh, Uzr, Uh, bzrh, Wmc, bmc):
    full = lambda *blk: pl.BlockSpec(blk, lambda i: (0,) * len(blk))
    return pl.pallas_call(
        _gru_msg_body,
        grid=(NB,),
        in_specs=[
            pl.BlockSpec((BLK, D), lambda i: (i, 0)),
            pl.BlockSpec((NC, BLK, D), lambda i: (0, i, 0)),
            full(D, 3 * D), full(D, 2 * D), full(D, D), full(1, 3 * D),
            full(D, T * D), full(1, T * D),
        ],
        out_specs=[
            pl.BlockSpec((BLK, D), lambda i: (i, 0)),
            pl.BlockSpec((T, BLK, D), lambda i: (0, i, 0)),
        ],
        out_shape=[
            jax.ShapeDtypeStruct((NP_, D), jnp.float32),
            jax.ShapeDtypeStruct((T, NP_, D), jnp.float32),
        ],
    )(h, aggs, Wzrh, Uzr, Uh, bzrh, Wmc, bmc)


def _gru_readout_body(h_ref, agg_ref, wzrh, uzr, uh, bzrh, wug, bug,
                      ids_ref, out_ref):
    hn = _gru(h_ref, agg_ref, wzrh, uzr, uh, bzrh)
    prod = jnp.dot(hn, wug[...], preferred_element_type=jnp.float32) + bug[...]
    y = jax.nn.sigmoid(prod[:, EMB:]) * prod[:, :EMB]
    ids = ids_ref[0, 0, :]
    onehot = (
        lax.broadcasted_iota(jnp.int32, (G, BLK), 0) == ids[None, :]
    ).astype(jnp.float32)
    contrib = jnp.dot(onehot, y, preferred_element_type=jnp.float32)

    @pl.when(pl.program_id(0) == 0)
    def _():
        out_ref[...] = jnp.zeros_like(out_ref)

    out_ref[...] += contrib


def _gru_readout(h, aggs, Wzrh, Uzr, Uh, bzrh, Wug, bug, ids):
    full = lambda *blk: pl.BlockSpec(blk, lambda i: (0,) * len(blk))
    return pl.pallas_call(
        _gru_readout_body,
        grid=(NB,),
        in_specs=[
            pl.BlockSpec((BLK, D), lambda i: (i, 0)),
            pl.BlockSpec((NC, BLK, D), lambda i: (0, i, 0)),
            full(D, 3 * D), full(D, 2 * D), full(D, D), full(1, 3 * D),
            full(D, 2 * EMB), full(1, 2 * EMB),
            pl.BlockSpec((1, 1, BLK), lambda i: (i, 0, 0)),
        ],
        out_specs=pl.BlockSpec((G, EMB), lambda i: (0, 0)),
        out_shape=jax.ShapeDtypeStruct((G, EMB), jnp.float32),
    )(h, aggs, Wzrh, Uzr, Uh, bzrh, Wug, bug, ids)


def kernel(node_features, edge_index, edge_type, node_to_graph_id,
           W_msg, b_msg, Wz, Uz, bz, Wr, Ur, br, Wh, Uh, bh,
           Wu, bu, Wg, bg):
    f32 = jnp.float32
    h0 = jnp.pad(node_features, ((0, NP_ - N), (0, 0)))
    pad_src = jnp.arange(EP - E, dtype=jnp.int32) % N
    pad_dst = N + jnp.arange(EP - E, dtype=jnp.int32) % (NP_ - N)
    srcp = jnp.concatenate([edge_index[0], pad_src])
    dstp = jnp.concatenate([edge_index[1], pad_dst])
    etp = jnp.pad(edge_type, (0, EP - E))

    fused = _edge_prep(
        srcp.reshape(EP // K, K), etp.reshape(EP // K, K)
    ).reshape(NCHT, K)
    dst2d = dstp.reshape(NCHT, K)
    idx_pack = jnp.stack([fused, dst2d], axis=1)
    zeros128 = jnp.zeros((K, D), f32)
    idsp = jnp.pad(
        node_to_graph_id, (0, NP_ - N), constant_values=G
    ).reshape(NB, 1, BLK)

    Wzrh = jnp.concatenate([Wz, Wr, Wh], axis=1)
    Uzr = jnp.concatenate([Uz, Ur], axis=1)
    bzrh = jnp.concatenate([bz, br, bh]).reshape(1, 3 * D)
    Wmc = jnp.moveaxis(W_msg, 0, 1).reshape(D, T * D)
    bmc = b_msg.reshape(1, T * D)
    Wug = jnp.concatenate([Wu, Wg], axis=1)
    bug = jnp.concatenate([bu, bg]).reshape(1, 2 * EMB)

    tr = _msg(h0, Wmc, bmc)
    h = h0
    for layer in range(L):
        aggs = _sc_scatter(tr.reshape(T * NP_, D), idx_pack, zeros128)
        if layer < L - 1:
            h, tr = _gru_msg(h, aggs, Wzrh, Uzr, Uh, bzrh, Wmc, bmc)
        else:
            out = _gru_readout(h, aggs, Wzrh, Uzr, Uh, bzrh, Wug, bug, idsp)
    return out


# speedup vs baseline: 1.1814x; 1.0150x over previous
import functools

import jax
import jax.numpy as jnp
from jax import lax
from jax.experimental import pallas as pl
from jax.experimental.pallas import tpu as pltpu
from jax.experimental.pallas import tpu_sc as plsc

N = 10000
E = 320000
D = 128
T = 3
G = 256
EMB = 512
L = 4

NP_ = 10240
BLK = 512
NB = NP_ // BLK
NC = 2
NS = 16
NW = NC * NS
K = 128
CHC = 80
NCHT = NW * CHC
EP = NCHT * K
RPT = NP_ // NS


def _prep_body(src_ref, et_ref, out_ref):
    out_ref[...] = et_ref[...] * NP_ + src_ref[...]


def _edge_prep(src2d, et2d):
    return pl.pallas_call(
        _prep_body,
        out_shape=jax.ShapeDtypeStruct(src2d.shape, jnp.int32),
    )(src2d, et2d)


def _msg_body(h_ref, w_ref, b_ref, out_ref):
    prod = (
        jnp.dot(h_ref[...], w_ref[...], preferred_element_type=jnp.float32)
        + b_ref[...]
    )
    for t in range(T):
        out_ref[t] = prod[:, t * D:(t + 1) * D]


def _msg(h, Wmc, bmc):
    return pl.pallas_call(
        _msg_body,
        grid=(NB,),
        in_specs=[
            pl.BlockSpec((BLK, D), lambda i: (i, 0)),
            pl.BlockSpec((D, T * D), lambda i: (0, 0)),
            pl.BlockSpec((1, T * D), lambda i: (0, 0)),
        ],
        out_specs=pl.BlockSpec((T, BLK, D), lambda i: (0, i, 0)),
        out_shape=jax.ShapeDtypeStruct((T, NP_, D), jnp.float32),
    )(h, Wmc, bmc)


def _sc_body(table, idx_hbm, zeros_hbm, out_hbm,
             idx_v, rows_v, agg_sh, sem_i0, sem_i1, sem_i2, sem_i3,
             sem_g0, sem_g1):
    c = lax.axis_index("c")
    s = lax.axis_index("s")
    base_chunk = c * (NS * CHC) + s
    sem_i = (sem_i0, sem_i1, sem_i2, sem_i3)
    sem_g = (sem_g0, sem_g1)

    base = s * RPT
    def start_idx(j, sl):
        pltpu.async_copy(idx_hbm.at[base_chunk + NS * j], idx_v.at[sl],
                         sem_i[sl])

    def wait_idx(sl):
        pltpu.make_async_copy(idx_hbm.at[0], idx_v.at[sl], sem_i[sl]).wait()

    def start_gather(b, sl):
        pltpu.async_copy(table.at[idx_v.at[sl, 0]], rows_v.at[b], sem_g[b])

    def wait_gather(b):
        pltpu.make_async_copy(table.at[pl.ds(0, K)], rows_v.at[b],
                              sem_g[b]).wait()

    def scatter(b, sl):
        pltpu.sync_copy(rows_v.at[b], agg_sh.at[idx_v.at[sl, 1]], add=True)

    with jax.named_scope("edge_loop_prime"):
        for j in range(4):
            start_idx(j, j)
        wait_idx(0)
        start_gather(0, 0)

    with jax.named_scope("agg_zero"):
        pltpu.sync_copy(zeros_hbm, rows_v.at[1])
        zdesc = []
        for k in range(RPT // K):
            zdesc.append(pltpu.async_copy(
                rows_v.at[1], agg_sh.at[pl.ds(base + k * K, K)], sem_i0))
        for dsc in zdesc:
            dsc.wait()
        plsc.subcore_barrier()

    wait_idx(1)
    start_gather(1, 1)

    @pl.loop(0, CHC, step=4)
    def _(g):
        wait_gather(0)
        scatter(0, 0)

        @pl.when(g + 4 < CHC)
        def _():
            start_idx(g + 4, 0)

        wait_idx(2)
        start_gather(0, 2)
        wait_gather(1)
        scatter(1, 1)

        @pl.when(g + 5 < CHC)
        def _():
            start_idx(g + 5, 1)

        wait_idx(3)
        start_gather(1, 3)
        wait_gather(0)
        scatter(0, 2)

        @pl.when(g + 6 < CHC)
        def _():
            start_idx(g + 6, 2)

        @pl.when(g + 4 < CHC)
        def _():
            wait_idx(0)
            start_gather(0, 0)

        wait_gather(1)
        scatter(1, 3)

        @pl.when(g + 7 < CHC)
        def _():
            start_idx(g + 7, 3)

        @pl.when(g + 5 < CHC)
        def _():
            wait_idx(1)
            start_gather(1, 1)

    with jax.named_scope("post_barrier"):
        plsc.subcore_barrier()

    with jax.named_scope("agg_writeout"):
        nw_sem = (sem_g0, sem_g1)
        for k in range(RPT // K):
            b = k % 2
            if k >= 2:
                pltpu.make_async_copy(
                    rows_v.at[b],
                    out_hbm.at[c, pl.ds(base + (k - 2) * K, K)],
                    nw_sem[b]).wait()
            pltpu.sync_copy(agg_sh.at[pl.ds(base + k * K, K)], rows_v.at[b])
            pltpu.async_copy(rows_v.at[b],
                             out_hbm.at[c, pl.ds(base + k * K, K)], nw_sem[b])
        for k in (RPT // K - 2, RPT // K - 1):
            pltpu.make_async_copy(rows_v.at[k % 2],
                                  out_hbm.at[c, pl.ds(base + k * K, K)],
                                  nw_sem[k % 2]).wait()


@functools.cache
def _make_sc_scatter():
    return pl.kernel(
        _sc_body,
        out_type=jax.ShapeDtypeStruct((NC, NP_, D), jnp.float32),
        mesh=plsc.VectorSubcoreMesh(core_axis_name="c", subcore_axis_name="s",
                                    num_cores=NC, num_subcores=NS),
        scratch_types=[
            pltpu.VMEM((4, 2, K), jnp.int32),
            pltpu.VMEM((2, K, D), jnp.float32),
            pltpu.VMEM_SHARED((NP_, D), jnp.float32),
        ] + [pltpu.SemaphoreType.DMA] * 6,
    )


def _sc_scatter(table, idx_pack, zeros128):
    return _make_sc_scatter()(table, idx_pack, zeros128)


def _gru(h_ref, agg_ref, wzrh, uzr, uh, bzrh):
    h = h_ref[...]
    a = agg_ref[...].sum(axis=0)
    dot = lambda x, m: jnp.dot(x, m, preferred_element_type=jnp.float32)
    za = dot(a, wzrh[...]) + bzrh[...]
    zh = dot(h, uzr[...])
    z = jax.nn.sigmoid(za[:, :D] + zh[:, :D])
    r = jax.nn.sigmoid(za[:, D:2 * D] + zh[:, D:2 * D])
    ht = jnp.tanh(za[:, 2 * D:] + dot(r * h, uh[...]))
    return (1.0 - z) * h + z * ht


def _gru_msg_body(h_ref, agg_ref, wzrh, uzr, uh, bzrh, wmc, bmc,
                  hn_ref, tr_ref):
    hn = _gru(h_ref, agg_ref, wzrh, uzr, uh, bzrh)
    hn_ref[...] = hn
    prod = jnp.dot(hn, wmc[...], preferred_element_type=jnp.float32) + bmc[...]
    for t in range(T):
        tr_ref[t] = prod[:, t * D:(t + 1) * D]


def _gru_msg(h, aggs, Wzrh, Uzr, Uh, bzrh, Wmc, bmc):
    full = lambda *blk: pl.BlockSpec(blk, lambda i: (0,) * len(blk))
    return pl.pallas_call(
        _gru_msg_body,
        grid=(NB,),
        in_specs=[
            pl.BlockSpec((BLK, D), lambda i: (i, 0)),
            pl.BlockSpec((NC, BLK, D), lambda i: (0, i, 0)),
            full(D, 3 * D), full(D, 2 * D), full(D, D), full(1, 3 * D),
            full(D, T * D), full(1, T * D),
        ],
        out_specs=[
            pl.BlockSpec((BLK, D), lambda i: (i, 0)),
            pl.BlockSpec((T, BLK, D), lambda i: (0, i, 0)),
        ],
        out_shape=[
            jax.ShapeDtypeStruct((NP_, D), jnp.float32),
            jax.ShapeDtypeStruct((T, NP_, D), jnp.float32),
        ],
    )(h, aggs, Wzrh, Uzr, Uh, bzrh, Wmc, bmc)


def _gru_readout_body(h_ref, agg_ref, wzrh, uzr, uh, bzrh, wug, bug,
                      ids_ref, out_ref):
    hn = _gru(h_ref, agg_ref, wzrh, uzr, uh, bzrh)
    prod = jnp.dot(hn, wug[...], preferred_element_type=jnp.float32) + bug[...]
    y = jax.nn.sigmoid(prod[:, EMB:]) * prod[:, :EMB]
    ids = ids_ref[0, 0, :]
    onehot = (
        lax.broadcasted_iota(jnp.int32, (G, BLK), 0) == ids[None, :]
    ).astype(jnp.float32)
    contrib = jnp.dot(onehot, y, preferred_element_type=jnp.float32)

    @pl.when(pl.program_id(0) == 0)
    def _():
        out_ref[...] = jnp.zeros_like(out_ref)

    out_ref[...] += contrib


def _gru_readout(h, aggs, Wzrh, Uzr, Uh, bzrh, Wug, bug, ids):
    full = lambda *blk: pl.BlockSpec(blk, lambda i: (0,) * len(blk))
    return pl.pallas_call(
        _gru_readout_body,
        grid=(NB,),
        in_specs=[
            pl.BlockSpec((BLK, D), lambda i: (i, 0)),
            pl.BlockSpec((NC, BLK, D), lambda i: (0, i, 0)),
            full(D, 3 * D), full(D, 2 * D), full(D, D), full(1, 3 * D),
            full(D, 2 * EMB), full(1, 2 * EMB),
            pl.BlockSpec((1, 1, BLK), lambda i: (i, 0, 0)),
        ],
        out_specs=pl.BlockSpec((G, EMB), lambda i: (0, 0)),
        out_shape=jax.ShapeDtypeStruct((G, EMB), jnp.float32),
    )(h, aggs, Wzrh, Uzr, Uh, bzrh, Wug, bug, ids)


def kernel(node_features, edge_index, edge_type, node_to_graph_id,
           W_msg, b_msg, Wz, Uz, bz, Wr, Ur, br, Wh, Uh, bh,
           Wu, bu, Wg, bg):
    f32 = jnp.float32
    h0 = jnp.pad(node_features, ((0, NP_ - N), (0, 0)))
    pad_src = jnp.arange(EP - E, dtype=jnp.int32) % N
    pad_dst = N + jnp.arange(EP - E, dtype=jnp.int32) % (NP_ - N)
    srcp = jnp.concatenate([edge_index[0], pad_src])
    dstp = jnp.concatenate([edge_index[1], pad_dst])
    etp = jnp.pad(edge_type, (0, EP - E))

    fused = _edge_prep(
        srcp.reshape(EP // K, K), etp.reshape(EP // K, K)
    ).reshape(NCHT, K)
    dst2d = dstp.reshape(NCHT, K)
    idx_pack = jnp.stack([fused, dst2d], axis=1)
    zeros128 = jnp.zeros((K, D), f32)
    idsp = jnp.pad(
        node_to_graph_id, (0, NP_ - N), constant_values=G
    ).reshape(NB, 1, BLK)

    Wzrh = jnp.concatenate([Wz, Wr, Wh], axis=1)
    Uzr = jnp.concatenate([Uz, Ur], axis=1)
    bzrh = jnp.concatenate([bz, br, bh]).reshape(1, 3 * D)
    Wmc = jnp.moveaxis(W_msg, 0, 1).reshape(D, T * D)
    bmc = b_msg.reshape(1, T * D)
    Wug = jnp.concatenate([Wu, Wg], axis=1)
    bug = jnp.concatenate([bu, bg]).reshape(1, 2 * EMB)

    tr = _msg(h0, Wmc, bmc)
    h = h0
    for layer in range(L):
        aggs = _sc_scatter(tr.reshape(T * NP_, D), idx_pack, zeros128)
        if layer < L - 1:
            h, tr = _gru_msg(h, aggs, Wzrh, Uzr, Uh, bzrh, Wmc, bmc)
        else:
            out = _gru_readout(h, aggs, Wzrh, Uzr, Uh, bzrh, Wug, bug, idsp)
    return out
